# Initial kernel scaffold; baseline (speedup 1.0000x reference)
import jax
import jax.numpy as jnp
from jax.experimental import pallas as pl


def kernel(z, codebook):
    raise NotImplementedError("write your pallas kernel here")


# speedup vs baseline: 1.3774x; 1.3774x over previous
import functools

import jax
import jax.numpy as jnp
from jax import lax
from jax.experimental import pallas as pl
from jax.experimental.pallas import tpu as pltpu
from jax.experimental.pallas import tpu_sc as plsc

_B, _T, _D = 64, 576, 64
_N = _B * _T
_K = 1024

_R = 512
_STEPS = _N // _R

_NC, _NS = 2, 16
_NW = _NC * _NS
_ROWS_PER_TILE = _N // _NW
_CHUNK = 128
_NCHUNK = _ROWS_PER_TILE // _CHUNK


def _tc_body(z_ref, cb_ref, idx_ref, loss_ref):
    i = pl.program_id(0)
    zb = z_ref[...]
    cb = cb_ref[...]
    rn = jnp.sum(zb * zb, axis=1, keepdims=True)
    cbn = jnp.sum(cb * cb, axis=1)[None, :]
    dots = lax.dot_general(zb, cb, (((1,), (1,)), ((), ())),
                           preferred_element_type=jnp.float32)
    d2 = rn - 2.0 * dots + cbn
    m = jnp.min(d2, axis=1)
    ids = lax.broadcasted_iota(jnp.int32, d2.shape, 1)
    idx = jnp.min(jnp.where(d2 == m[:, None], ids, _K), axis=1)
    idx_ref[0, 0, :] = idx
    part = jnp.sum(m)

    @pl.when(i == 0)
    def _init():
        loss_ref[0, 0] = 0.0

    loss_ref[0, 0] += part

    @pl.when(i == _STEPS - 1)
    def _finish():
        loss_ref[0, 0] = loss_ref[0, 0] * (1.25 / (_N * _D))


_tc_call = pl.pallas_call(
    _tc_body,
    grid=(_STEPS,),
    in_specs=[
        pl.BlockSpec((_R, _D), lambda i: (i, 0)),
        pl.BlockSpec((_K, _D), lambda i: (0, 0)),
    ],
    out_specs=[
        pl.BlockSpec((1, 1, _R), lambda i: (i, 0, 0)),
        pl.BlockSpec(memory_space=pltpu.SMEM, block_shape=(1, 1),
                     index_map=lambda i: (0, 0)),
    ],
    out_shape=[
        jax.ShapeDtypeStruct((_STEPS, 1, _R), jnp.int32),
        jax.ShapeDtypeStruct((1, 1), jnp.float32),
    ],
)


@functools.cache
def _make_sc_gather():
    mesh = plsc.VectorSubcoreMesh(core_axis_name="c", subcore_axis_name="s")

    @functools.partial(
        pl.kernel,
        mesh=mesh,
        out_type=jax.ShapeDtypeStruct((_N, _D), jnp.float32),
        scratch_types=[
            pltpu.VMEM((_ROWS_PER_TILE,), jnp.int32),
            pltpu.VMEM((_ROWS_PER_TILE, _D), jnp.float32),
            pltpu.SemaphoreType.DMA,
        ],
        compiler_params=pltpu.CompilerParams(use_tc_tiling_on_sc=False),
    )
    def _sc_gather(cb_hbm, idx_hbm, out_hbm, idx_v, rows_v, sem):
        wid = lax.axis_index("s") * _NC + lax.axis_index("c")
        base = wid * _ROWS_PER_TILE
        pltpu.sync_copy(idx_hbm.at[pl.ds(base, _ROWS_PER_TILE)], idx_v)
        copies = [
            pltpu.async_copy(
                cb_hbm.at[idx_v.at[pl.ds(c * _CHUNK, _CHUNK)]],
                rows_v.at[pl.ds(c * _CHUNK, _CHUNK), :],
                sem,
            )
            for c in range(_NCHUNK)
        ]
        for cp in copies:
            cp.wait()
        pltpu.sync_copy(rows_v, out_hbm.at[pl.ds(base, _ROWS_PER_TILE)])

    return _sc_gather


def kernel(z, codebook):
    B, T, D = z.shape
    flat = z.reshape(_N, D)
    idx3, loss = _tc_call(flat, codebook)
    idx1 = idx3.reshape(_N)
    q = _make_sc_gather()(codebook, idx1)
    return q.reshape(B, T, D), loss.reshape(()), idx1.reshape(B, T)


# speedup vs baseline: 1.4955x; 1.0858x over previous
import functools

import jax
import jax.numpy as jnp
from jax import lax
from jax.experimental import pallas as pl
from jax.experimental.pallas import tpu as pltpu
from jax.experimental.pallas import tpu_sc as plsc

_B, _T, _D = 64, 576, 64
_N = _B * _T
_K = 1024

_R = 512
_STEPS = _N // _R

_NC, _NS = 2, 16
_NW = _NC * _NS
_ROWS_PER_TILE = _N // _NW
_CHUNK = 128
_NCHUNK = _ROWS_PER_TILE // _CHUNK


def _tc_body(z_ref, cb_ref, idx_ref, loss_ref, cbn_ref, ids_ref):
    i = pl.program_id(0)
    zb = z_ref[...]
    rn = jnp.sum(zb * zb, axis=1, keepdims=True)

    @pl.when(i == 0)
    def _init():
        cb = cb_ref[...]
        cbn_ref[...] = jnp.sum(cb * cb, axis=1)[None, :]
        ids_ref[...] = lax.broadcasted_iota(
            jnp.int32, (_R, _K), 1).astype(jnp.float32)
        loss_ref[0, 0] = 0.0

    dots = lax.dot_general(zb, cb_ref[...], (((1,), (1,)), ((), ())),
                           preferred_element_type=jnp.float32)
    d2 = rn - 2.0 * dots + cbn_ref[...]
    m = jnp.min(d2, axis=1)
    idxf = jnp.min(jnp.where(d2 == m[:, None], ids_ref[...], float(_K)), axis=1)
    idx_ref[0, 0, :] = idxf.astype(jnp.int32)
    loss_ref[0, 0] += jnp.sum(m)

    @pl.when(i == _STEPS - 1)
    def _finish():
        loss_ref[0, 0] = loss_ref[0, 0] * (1.25 / (_N * _D))


_tc_call = pl.pallas_call(
    _tc_body,
    grid=(_STEPS,),
    in_specs=[
        pl.BlockSpec((_R, _D), lambda i: (i, 0)),
        pl.BlockSpec((_K, _D), lambda i: (0, 0)),
    ],
    out_specs=[
        pl.BlockSpec((1, 1, _R), lambda i: (i, 0, 0)),
        pl.BlockSpec(memory_space=pltpu.SMEM, block_shape=(1, 1),
                     index_map=lambda i: (0, 0)),
    ],
    out_shape=[
        jax.ShapeDtypeStruct((_STEPS, 1, _R), jnp.int32),
        jax.ShapeDtypeStruct((1, 1), jnp.float32),
    ],
    scratch_shapes=[pltpu.VMEM((1, _K), jnp.float32),
                    pltpu.VMEM((_R, _K), jnp.float32)],
)


@functools.cache
def _make_sc_gather():
    mesh = plsc.VectorSubcoreMesh(core_axis_name="c", subcore_axis_name="s")

    @functools.partial(
        pl.kernel,
        mesh=mesh,
        out_type=jax.ShapeDtypeStruct((_N, _D), jnp.float32),
        scratch_types=[
            pltpu.VMEM((_ROWS_PER_TILE,), jnp.int32),
            pltpu.VMEM((_ROWS_PER_TILE, _D), jnp.float32),
            pltpu.SemaphoreType.DMA,
        ],
        compiler_params=pltpu.CompilerParams(use_tc_tiling_on_sc=False),
    )
    def _sc_gather(cb_hbm, idx_hbm, out_hbm, idx_v, rows_v, sem):
        wid = lax.axis_index("s") * _NC + lax.axis_index("c")
        base = wid * _ROWS_PER_TILE
        pltpu.sync_copy(idx_hbm.at[pl.ds(base, _ROWS_PER_TILE)], idx_v)
        copies = [
            pltpu.async_copy(
                cb_hbm.at[idx_v.at[pl.ds(c * _CHUNK, _CHUNK)]],
                rows_v.at[pl.ds(c * _CHUNK, _CHUNK), :],
                sem,
            )
            for c in range(_NCHUNK)
        ]
        for cp in copies:
            cp.wait()
        pltpu.sync_copy(rows_v, out_hbm.at[pl.ds(base, _ROWS_PER_TILE)])

    return _sc_gather


def kernel(z, codebook):
    B, T, D = z.shape
    flat = z.reshape(_N, D)
    idx3, loss = _tc_call(flat, codebook)
    idx1 = idx3.reshape(_N)
    q = _make_sc_gather()(codebook, idx1)
    return q.reshape(B, T, D), loss.reshape(()), idx1.reshape(B, T)


# speedup vs baseline: 1.5293x; 1.0226x over previous
import functools

import jax
import jax.numpy as jnp
from jax import lax
from jax.experimental import pallas as pl
from jax.experimental.pallas import tpu as pltpu
from jax.experimental.pallas import tpu_sc as plsc

_B, _T, _D = 64, 576, 64
_N = _B * _T
_K = 1024

_R = 1024
_STEPS = _N // _R

_NC, _NS = 2, 16
_NW = _NC * _NS
_ROWS_PER_TILE = _N // _NW
_CHUNK = 128
_NCHUNK = _ROWS_PER_TILE // _CHUNK


def _tc_body(z_ref, cb_ref, idx_ref, loss_ref, cbn_ref, ids_ref):
    i = pl.program_id(0)
    zb = z_ref[...]
    rn = jnp.sum(zb * zb, axis=1, keepdims=True)

    @pl.when(i == 0)
    def _init():
        cb = cb_ref[...]
        cbn_ref[...] = jnp.sum(cb * cb, axis=1)[None, :]
        ids_ref[...] = lax.broadcasted_iota(
            jnp.int32, (_R, _K), 1).astype(jnp.float32)
        loss_ref[0, 0] = 0.0

    dots = lax.dot_general(zb, cb_ref[...], (((1,), (1,)), ((), ())),
                           preferred_element_type=jnp.float32)
    d2 = rn - 2.0 * dots + cbn_ref[...]
    m = jnp.min(d2, axis=1)
    idxf = jnp.min(jnp.where(d2 == m[:, None], ids_ref[...], float(_K)), axis=1)
    idx_ref[0, 0, :] = idxf.astype(jnp.int32)
    loss_ref[0, 0] += jnp.sum(m)

    @pl.when(i == _STEPS - 1)
    def _finish():
        loss_ref[0, 0] = loss_ref[0, 0] * (1.25 / (_N * _D))


_tc_call = pl.pallas_call(
    _tc_body,
    grid=(_STEPS,),
    in_specs=[
        pl.BlockSpec((_R, _D), lambda i: (i, 0)),
        pl.BlockSpec((_K, _D), lambda i: (0, 0)),
    ],
    out_specs=[
        pl.BlockSpec((1, 1, _R), lambda i: (i, 0, 0)),
        pl.BlockSpec(memory_space=pltpu.SMEM, block_shape=(1, 1),
                     index_map=lambda i: (0, 0)),
    ],
    out_shape=[
        jax.ShapeDtypeStruct((_STEPS, 1, _R), jnp.int32),
        jax.ShapeDtypeStruct((1, 1), jnp.float32),
    ],
    scratch_shapes=[pltpu.VMEM((1, _K), jnp.float32),
                    pltpu.VMEM((_R, _K), jnp.float32)],
)


@functools.cache
def _make_sc_gather():
    mesh = plsc.VectorSubcoreMesh(core_axis_name="c", subcore_axis_name="s")

    @functools.partial(
        pl.kernel,
        mesh=mesh,
        out_type=jax.ShapeDtypeStruct((_N, _D), jnp.float32),
        scratch_types=[
            pltpu.VMEM((_ROWS_PER_TILE,), jnp.int32),
            pltpu.VMEM((_ROWS_PER_TILE, _D), jnp.float32),
            pltpu.SemaphoreType.DMA,
        ],
        compiler_params=pltpu.CompilerParams(use_tc_tiling_on_sc=False),
    )
    def _sc_gather(cb_hbm, idx_hbm, out_hbm, idx_v, rows_v, sem):
        wid = lax.axis_index("s") * _NC + lax.axis_index("c")
        base = wid * _ROWS_PER_TILE
        pltpu.sync_copy(idx_hbm.at[pl.ds(base, _ROWS_PER_TILE)], idx_v)
        copies = [
            pltpu.async_copy(
                cb_hbm.at[idx_v.at[pl.ds(c * _CHUNK, _CHUNK)]],
                rows_v.at[pl.ds(c * _CHUNK, _CHUNK), :],
                sem,
            )
            for c in range(_NCHUNK)
        ]
        for cp in copies:
            cp.wait()
        pltpu.sync_copy(rows_v, out_hbm.at[pl.ds(base, _ROWS_PER_TILE)])

    return _sc_gather


def kernel(z, codebook):
    B, T, D = z.shape
    flat = z.reshape(_N, D)
    idx3, loss = _tc_call(flat, codebook)
    idx1 = idx3.reshape(_N)
    q = _make_sc_gather()(codebook, idx1)
    return q.reshape(B, T, D), loss.reshape(()), idx1.reshape(B, T)


# speedup vs baseline: 1.5326x; 1.0022x over previous
import functools

import jax
import jax.numpy as jnp
from jax import lax
from jax.experimental import pallas as pl
from jax.experimental.pallas import tpu as pltpu
from jax.experimental.pallas import tpu_sc as plsc

_B, _T, _D = 64, 576, 64
_N = _B * _T
_K = 1024

_R = 1024
_STEPS = _N // _R

_NC, _NS = 2, 16
_NW = _NC * _NS
_ROWS_PER_TILE = _N // _NW
_CHUNK = 128
_NCHUNK = _ROWS_PER_TILE // _CHUNK


def _tc_body(z_ref, cb_ref, idx_ref, loss_ref, cbn_ref, ids_ref):
    i = pl.program_id(0)
    zb = z_ref[...]
    rn = jnp.sum(zb * zb, axis=1, keepdims=True)

    @pl.when(i == 0)
    def _init():
        cb = cb_ref[...]
        cbn_ref[...] = jnp.sum(cb * cb, axis=1)[None, :]
        ids_ref[...] = lax.broadcasted_iota(
            jnp.int32, (_R, _K), 1).astype(jnp.float32)
        loss_ref[0, 0] = 0.0

    dots = lax.dot_general(zb, cb_ref[...], (((1,), (1,)), ((), ())),
                           preferred_element_type=jnp.float32)
    d2 = rn - 2.0 * dots + cbn_ref[...]
    m = jnp.min(d2, axis=1)
    idxf = jnp.min(jnp.where(d2 == m[:, None], ids_ref[...], float(_K)), axis=1)
    idx_ref[...] = idxf.astype(jnp.int32)
    loss_ref[0, 0] += jnp.sum(m)

    @pl.when(i == _STEPS - 1)
    def _finish():
        loss_ref[0, 0] = loss_ref[0, 0] * (1.25 / (_N * _D))


_tc_call = pl.pallas_call(
    _tc_body,
    grid=(_STEPS,),
    in_specs=[
        pl.BlockSpec((_R, _D), lambda i: (i, 0)),
        pl.BlockSpec((_K, _D), lambda i: (0, 0)),
    ],
    out_specs=[
        pl.BlockSpec((_R,), lambda i: (i,)),
        pl.BlockSpec(memory_space=pltpu.SMEM, block_shape=(1, 1),
                     index_map=lambda i: (0, 0)),
    ],
    out_shape=[
        jax.ShapeDtypeStruct((_N,), jnp.int32),
        jax.ShapeDtypeStruct((1, 1), jnp.float32),
    ],
    scratch_shapes=[pltpu.VMEM((1, _K), jnp.float32),
                    pltpu.VMEM((_R, _K), jnp.float32)],
)


@functools.cache
def _make_sc_gather():
    mesh = plsc.VectorSubcoreMesh(core_axis_name="c", subcore_axis_name="s")

    @functools.partial(
        pl.kernel,
        mesh=mesh,
        out_type=jax.ShapeDtypeStruct((_N, _D), jnp.float32),
        scratch_types=[
            pltpu.VMEM((_ROWS_PER_TILE,), jnp.int32),
            pltpu.VMEM((_ROWS_PER_TILE, _D), jnp.float32),
            pltpu.SemaphoreType.DMA,
        ],
        compiler_params=pltpu.CompilerParams(use_tc_tiling_on_sc=False),
    )
    def _sc_gather(cb_hbm, idx_hbm, out_hbm, idx_v, rows_v, sem):
        wid = lax.axis_index("s") * _NC + lax.axis_index("c")
        base = wid * _ROWS_PER_TILE
        pltpu.sync_copy(idx_hbm.at[pl.ds(base, _ROWS_PER_TILE)], idx_v)
        copies = [
            pltpu.async_copy(
                cb_hbm.at[idx_v.at[pl.ds(c * _CHUNK, _CHUNK)]],
                rows_v.at[pl.ds(c * _CHUNK, _CHUNK), :],
                sem,
            )
            for c in range(_NCHUNK)
        ]
        for cp in copies:
            cp.wait()
        pltpu.sync_copy(rows_v, out_hbm.at[pl.ds(base, _ROWS_PER_TILE)])

    return _sc_gather


def kernel(z, codebook):
    B, T, D = z.shape
    flat = z.reshape(_N, D)
    idx1, loss = _tc_call(flat, codebook)
    q = _make_sc_gather()(codebook, idx1)
    return q.reshape(B, T, D), loss.reshape(()), idx1.reshape(B, T)
